# baseline (speedup 1.0000x reference)
import jax
import jax.numpy as jnp
from jax import lax
from jax.experimental import pallas as pl
from jax.experimental.pallas import tpu as pltpu
from jax.experimental.pallas import tpu_sc as plsc

NUM_CLASSES = 100000
HD = 128
N = 16384

NC = 2
NS = 16

BLOCK = 12800
NB = 4
DUMMY = BLOCK
ACC_ROWS = BLOCK + 8
LPT = N // NS
RPT = BLOCK // NS
CH = 64
NCH = LPT // CH
NRING = 3


def _body(labels_hbm, hv_hbm, w_hbm, out_hbm,
          labels_v, stage, dstidx, acc, sem_i, sem_o, sem_l, sem_a):
    c = lax.axis_index("c")
    s = lax.axis_index("s")
    lab_base = s * LPT

    pltpu.sync_copy(labels_hbm.at[pl.ds(lab_base, LPT)], labels_v)

    def blo_of(b):
        return (c * NB + b) * BLOCK

    def fire_init(b):
        row0 = blo_of(b) + s * RPT

        @pl.when(row0 < NUM_CLASSES)
        def _():
            pltpu.async_copy(
                w_hbm.at[pl.ds(row0, RPT)], acc.at[pl.ds(s * RPT, RPT)], sem_i
            )

    def wait_init(b):
        row0 = blo_of(b) + s * RPT

        @pl.when(row0 < NUM_CLASSES)
        def _():
            pltpu.make_async_copy(
                w_hbm.at[pl.ds(row0, RPT)], acc.at[pl.ds(s * RPT, RPT)], sem_i
            ).wait()

    def fire_load(k):
        r = k % NRING
        pltpu.async_copy(
            hv_hbm.at[pl.ds(lab_base + k * CH, CH)], stage.at[r], sem_l[r]
        )

    def wait_load(k):
        r = k % NRING
        pltpu.make_async_copy(
            hv_hbm.at[pl.ds(lab_base + k * CH, CH)], stage.at[r], sem_l[r]
        ).wait()

    def fire_add(k):
        r = k % NRING
        pltpu.async_copy(stage.at[r], acc.at[dstidx.at[r]], sem_a[r], add=True)

    def wait_add(k):
        r = k % NRING
        pltpu.make_async_copy(
            stage.at[r], acc.at[dstidx.at[r]], sem_a[r]
        ).wait()

    fire_init(0)

    for b in range(NB):
        blo = blo_of(b)
        bhi = blo + BLOCK

        wait_init(b)
        plsc.subcore_barrier()

        fire_load(0)
        fire_load(1)
        for k in range(NCH):
            r = k % NRING
            for g in range(CH // 16):
                lab = labels_v[pl.ds(k * CH + g * 16, 16)]
                in_blk = (lab >= blo) & (lab < bhi)
                dstidx[r, pl.ds(g * 16, 16)] = jnp.where(in_blk, lab - blo, DUMMY)
            wait_load(k)
            fire_add(k)
            if k >= 1:
                wait_add(k - 1)
            if k + 2 < NCH:
                fire_load(k + 2)
        wait_add(NCH - 1)

        plsc.subcore_barrier()

        row0 = blo + s * RPT

        @pl.when(row0 < NUM_CLASSES)
        def _():
            pltpu.async_copy(
                acc.at[pl.ds(s * RPT, RPT)], out_hbm.at[pl.ds(row0, RPT)], sem_o
            )

        @pl.when(row0 < NUM_CLASSES)
        def _():
            pltpu.make_async_copy(
                acc.at[pl.ds(s * RPT, RPT)], out_hbm.at[pl.ds(row0, RPT)], sem_o
            ).wait()

        if b + 1 < NB:
            fire_init(b + 1)


@jax.jit
def _scatter_add(labels, hv, classify_weights):
    mesh = plsc.VectorSubcoreMesh(
        core_axis_name="c", subcore_axis_name="s", num_cores=NC, num_subcores=NS
    )
    return pl.kernel(
        _body,
        out_type=jax.ShapeDtypeStruct((NUM_CLASSES, HD), jnp.float32),
        mesh=mesh,
        compiler_params=pltpu.CompilerParams(needs_layout_passes=False),
        scratch_types=[
            pltpu.VMEM((LPT,), jnp.int32),
            pltpu.VMEM((NRING, CH, HD), jnp.float32),
            pltpu.VMEM((NRING, CH), jnp.int32),
            pltpu.VMEM_SHARED((ACC_ROWS, HD), jnp.float32),
            pltpu.SemaphoreType.DMA,
            pltpu.SemaphoreType.DMA,
            [pltpu.SemaphoreType.DMA] * NRING,
            [pltpu.SemaphoreType.DMA] * NRING,
        ],
    )(labels, hv, classify_weights)


def kernel(labels, hv, classify_weights):
    return _scatter_add(labels, hv, classify_weights)


# speedup vs baseline: 1.2793x; 1.2793x over previous
import jax
import jax.numpy as jnp
from jax import lax
from jax.experimental import pallas as pl
from jax.experimental.pallas import tpu as pltpu
from jax.experimental.pallas import tpu_sc as plsc

NUM_CLASSES = 100000
HD = 128
N = 16384

NC = 2
NS = 16

BLOCK = 12800
NB = 4
DUMMY = BLOCK
ACC_ROWS = BLOCK + 16
LPT = N // NS
RPT = BLOCK // NS
CH = 64
NCH = LPT // CH
NRING = 3


def _body(labels_hbm, hv_hbm, w_hbm, out_hbm,
          labels_v, stage, dstidx, acc, sem_i, sem_o, sem_l, sem_a):
    c = lax.axis_index("c")
    s = lax.axis_index("s")
    lab_base = s * LPT

    pltpu.sync_copy(labels_hbm.at[pl.ds(lab_base, LPT)], labels_v)

    def blo_of(b):
        return (c * NB + b) * BLOCK

    def fire_init(b):
        row0 = blo_of(b) + s * RPT

        @pl.when(row0 < NUM_CLASSES)
        def _():
            pltpu.async_copy(
                w_hbm.at[pl.ds(row0, RPT)], acc.at[pl.ds(s * RPT, RPT)], sem_i
            )

    def wait_init(b):
        row0 = blo_of(b) + s * RPT

        @pl.when(row0 < NUM_CLASSES)
        def _():
            pltpu.make_async_copy(
                w_hbm.at[pl.ds(row0, RPT)], acc.at[pl.ds(s * RPT, RPT)], sem_i
            ).wait()

    def fire_load(k):
        r = k % NRING
        pltpu.async_copy(
            hv_hbm.at[pl.ds(lab_base + k * CH, CH)], stage.at[r], sem_l[r]
        )

    def wait_load(k):
        r = k % NRING
        pltpu.make_async_copy(
            hv_hbm.at[pl.ds(lab_base + k * CH, CH)], stage.at[r], sem_l[r]
        ).wait()

    def fire_add(k):
        r = k % NRING
        pltpu.async_copy(stage.at[r], acc.at[dstidx.at[r]], sem_a[r], add=True)

    def wait_add(k):
        r = k % NRING
        pltpu.make_async_copy(
            stage.at[r], acc.at[dstidx.at[r]], sem_a[r]
        ).wait()

    fire_init(0)

    for b in range(NB):
        blo = blo_of(b)
        bhi = blo + BLOCK

        wait_init(b)
        plsc.subcore_barrier()

        fire_load(0)
        fire_load(1)
        dummy_spread = DUMMY + lax.iota(jnp.int32, 16)
        for k in range(NCH):
            r = k % NRING
            for g in range(CH // 16):
                lab = labels_v[pl.ds(k * CH + g * 16, 16)]
                in_blk = (lab >= blo) & (lab < bhi)
                dstidx[r, pl.ds(g * 16, 16)] = jnp.where(
                    in_blk, lab - blo, dummy_spread
                )
            wait_load(k)
            fire_add(k)
            if k >= 1:
                wait_add(k - 1)
            if k + 2 < NCH:
                fire_load(k + 2)
        wait_add(NCH - 1)

        plsc.subcore_barrier()

        row0 = blo + s * RPT

        @pl.when(row0 < NUM_CLASSES)
        def _():
            pltpu.async_copy(
                acc.at[pl.ds(s * RPT, RPT)], out_hbm.at[pl.ds(row0, RPT)], sem_o
            )

        @pl.when(row0 < NUM_CLASSES)
        def _():
            pltpu.make_async_copy(
                acc.at[pl.ds(s * RPT, RPT)], out_hbm.at[pl.ds(row0, RPT)], sem_o
            ).wait()

        if b + 1 < NB:
            fire_init(b + 1)


@jax.jit
def _scatter_add(labels, hv, classify_weights):
    mesh = plsc.VectorSubcoreMesh(
        core_axis_name="c", subcore_axis_name="s", num_cores=NC, num_subcores=NS
    )
    return pl.kernel(
        _body,
        out_type=jax.ShapeDtypeStruct((NUM_CLASSES, HD), jnp.float32),
        mesh=mesh,
        compiler_params=pltpu.CompilerParams(needs_layout_passes=False),
        scratch_types=[
            pltpu.VMEM((LPT,), jnp.int32),
            pltpu.VMEM((NRING, CH, HD), jnp.float32),
            pltpu.VMEM((NRING, CH), jnp.int32),
            pltpu.VMEM_SHARED((ACC_ROWS, HD), jnp.float32),
            pltpu.SemaphoreType.DMA,
            pltpu.SemaphoreType.DMA,
            [pltpu.SemaphoreType.DMA] * NRING,
            [pltpu.SemaphoreType.DMA] * NRING,
        ],
    )(labels, hv, classify_weights)


def kernel(labels, hv, classify_weights):
    return _scatter_add(labels, hv, classify_weights)


# speedup vs baseline: 1.2840x; 1.0037x over previous
import jax
import jax.numpy as jnp
from jax import lax
from jax.experimental import pallas as pl
from jax.experimental.pallas import tpu as pltpu
from jax.experimental.pallas import tpu_sc as plsc

NUM_CLASSES = 100000
HD = 128
N = 16384

NC = 2
NS = 16

BLOCK = 12800
NB = 4
DUMMY = BLOCK
ACC_ROWS = BLOCK + 8
LPT = N // NS
RPT = BLOCK // NS
CH = 64
NCH = LPT // CH
NRING = 3


def _body(labels_hbm, hv_hbm, w_hbm, out_hbm,
          labels_v, stage, dstidx, acc, sem_i, sem_o, sem_l, sem_a):
    c = lax.axis_index("c")
    s = lax.axis_index("s")
    lab_base = s * LPT

    pltpu.sync_copy(labels_hbm.at[pl.ds(lab_base, LPT)], labels_v)

    def blo_of(b):
        return (c * NB + b) * BLOCK

    def fire_init(b):
        row0 = blo_of(b) + s * RPT

        @pl.when(row0 < NUM_CLASSES)
        def _():
            pltpu.async_copy(
                w_hbm.at[pl.ds(row0, RPT)], acc.at[pl.ds(s * RPT, RPT)], sem_i
            )

    def wait_init(b):
        row0 = blo_of(b) + s * RPT

        @pl.when(row0 < NUM_CLASSES)
        def _():
            pltpu.make_async_copy(
                w_hbm.at[pl.ds(row0, RPT)], acc.at[pl.ds(s * RPT, RPT)], sem_i
            ).wait()

    def fire_load(k):
        r = k % NRING
        pltpu.async_copy(
            hv_hbm.at[pl.ds(lab_base + k * CH, CH)], stage.at[r], sem_l[r]
        )

    def wait_load(k):
        r = k % NRING
        pltpu.make_async_copy(
            hv_hbm.at[pl.ds(lab_base + k * CH, CH)], stage.at[r], sem_l[r]
        ).wait()

    def fire_add(k):
        r = k % NRING
        pltpu.async_copy(stage.at[r], acc.at[dstidx.at[r]], sem_a[r], add=True)

    def wait_add(k):
        r = k % NRING
        pltpu.make_async_copy(
            stage.at[r], acc.at[dstidx.at[r]], sem_a[r]
        ).wait()

    fire_init(0)

    for b in range(NB):
        blo = blo_of(b)
        bhi = blo + BLOCK

        wait_init(b)
        plsc.subcore_barrier()

        fire_load(0)
        fire_load(1)
        dummy_spread = DUMMY + (lax.iota(jnp.int32, 16) & 7)
        for k in range(NCH):
            r = k % NRING
            for g in range(CH // 16):
                lab = labels_v[pl.ds(k * CH + g * 16, 16)]
                in_blk = (lab >= blo) & (lab < bhi)
                dstidx[r, pl.ds(g * 16, 16)] = jnp.where(
                    in_blk, lab - blo, dummy_spread
                )
            wait_load(k)
            fire_add(k)
            if k >= 1:
                wait_add(k - 1)
            if k + 2 < NCH:
                fire_load(k + 2)
        wait_add(NCH - 1)

        plsc.subcore_barrier()

        row0 = blo + s * RPT

        @pl.when(row0 < NUM_CLASSES)
        def _():
            pltpu.async_copy(
                acc.at[pl.ds(s * RPT, RPT)], out_hbm.at[pl.ds(row0, RPT)], sem_o
            )

        @pl.when(row0 < NUM_CLASSES)
        def _():
            pltpu.make_async_copy(
                acc.at[pl.ds(s * RPT, RPT)], out_hbm.at[pl.ds(row0, RPT)], sem_o
            ).wait()

        if b + 1 < NB:
            fire_init(b + 1)


@jax.jit
def _scatter_add(labels, hv, classify_weights):
    mesh = plsc.VectorSubcoreMesh(
        core_axis_name="c", subcore_axis_name="s", num_cores=NC, num_subcores=NS
    )
    return pl.kernel(
        _body,
        out_type=jax.ShapeDtypeStruct((NUM_CLASSES, HD), jnp.float32),
        mesh=mesh,
        compiler_params=pltpu.CompilerParams(needs_layout_passes=False),
        scratch_types=[
            pltpu.VMEM((LPT,), jnp.int32),
            pltpu.VMEM((NRING, CH, HD), jnp.float32),
            pltpu.VMEM((NRING, CH), jnp.int32),
            pltpu.VMEM_SHARED((ACC_ROWS, HD), jnp.float32),
            pltpu.SemaphoreType.DMA,
            pltpu.SemaphoreType.DMA,
            [pltpu.SemaphoreType.DMA] * NRING,
            [pltpu.SemaphoreType.DMA] * NRING,
        ],
    )(labels, hv, classify_weights)


def kernel(labels, hv, classify_weights):
    return _scatter_add(labels, hv, classify_weights)


# speedup vs baseline: 1.2863x; 1.0018x over previous
import jax
import jax.numpy as jnp
from jax import lax
from jax.experimental import pallas as pl
from jax.experimental.pallas import tpu as pltpu
from jax.experimental.pallas import tpu_sc as plsc

NUM_CLASSES = 100000
HD = 128
N = 16384

NC = 2
NS = 16

BLOCK = 12800
NB = 4
DUMMY = BLOCK
ACC_ROWS = BLOCK + 256
LPT = N // NS
RPT = BLOCK // NS
CH = 64
NCH = LPT // CH
NRING = 3


def _body(labels_hbm, hv_hbm, w_hbm, out_hbm,
          labels_v, stage, dstidx, acc, sem_i, sem_o, sem_l, sem_a):
    c = lax.axis_index("c")
    s = lax.axis_index("s")
    lab_base = s * LPT

    pltpu.sync_copy(labels_hbm.at[pl.ds(lab_base, LPT)], labels_v)

    def blo_of(b):
        return (c * NB + b) * BLOCK

    def fire_init(b):
        row0 = blo_of(b) + s * RPT

        @pl.when(row0 < NUM_CLASSES)
        def _():
            pltpu.async_copy(
                w_hbm.at[pl.ds(row0, RPT)], acc.at[pl.ds(s * RPT, RPT)], sem_i
            )

    def wait_init(b):
        row0 = blo_of(b) + s * RPT

        @pl.when(row0 < NUM_CLASSES)
        def _():
            pltpu.make_async_copy(
                w_hbm.at[pl.ds(row0, RPT)], acc.at[pl.ds(s * RPT, RPT)], sem_i
            ).wait()

    def fire_load(k):
        r = k % NRING
        pltpu.async_copy(
            hv_hbm.at[pl.ds(lab_base + k * CH, CH)], stage.at[r], sem_l[r]
        )

    def wait_load(k):
        r = k % NRING
        pltpu.make_async_copy(
            hv_hbm.at[pl.ds(lab_base + k * CH, CH)], stage.at[r], sem_l[r]
        ).wait()

    def fire_add(k):
        r = k % NRING
        pltpu.async_copy(stage.at[r], acc.at[dstidx.at[r]], sem_a[r], add=True)

    def wait_add(k):
        r = k % NRING
        pltpu.make_async_copy(
            stage.at[r], acc.at[dstidx.at[r]], sem_a[r]
        ).wait()

    fire_init(0)

    for b in range(NB):
        blo = blo_of(b)
        bhi = blo + BLOCK

        wait_init(b)
        plsc.subcore_barrier()

        fire_load(0)
        fire_load(1)

        for k in range(NCH):
            r = k % NRING
            for g in range(CH // 16):
                lab = labels_v[pl.ds(k * CH + g * 16, 16)]
                in_blk = (lab >= blo) & (lab < bhi)
                dstidx[r, pl.ds(g * 16, 16)] = jnp.where(
                    in_blk, lab - blo,
                    DUMMY + ((k % 16) * 16 + lax.iota(jnp.int32, 16))
                )
            wait_load(k)
            fire_add(k)
            if k >= 1:
                wait_add(k - 1)
            if k + 2 < NCH:
                fire_load(k + 2)
        wait_add(NCH - 1)

        plsc.subcore_barrier()

        row0 = blo + s * RPT

        @pl.when(row0 < NUM_CLASSES)
        def _():
            pltpu.async_copy(
                acc.at[pl.ds(s * RPT, RPT)], out_hbm.at[pl.ds(row0, RPT)], sem_o
            )

        @pl.when(row0 < NUM_CLASSES)
        def _():
            pltpu.make_async_copy(
                acc.at[pl.ds(s * RPT, RPT)], out_hbm.at[pl.ds(row0, RPT)], sem_o
            ).wait()

        if b + 1 < NB:
            fire_init(b + 1)


@jax.jit
def _scatter_add(labels, hv, classify_weights):
    mesh = plsc.VectorSubcoreMesh(
        core_axis_name="c", subcore_axis_name="s", num_cores=NC, num_subcores=NS
    )
    return pl.kernel(
        _body,
        out_type=jax.ShapeDtypeStruct((NUM_CLASSES, HD), jnp.float32),
        mesh=mesh,
        compiler_params=pltpu.CompilerParams(needs_layout_passes=False),
        scratch_types=[
            pltpu.VMEM((LPT,), jnp.int32),
            pltpu.VMEM((NRING, CH, HD), jnp.float32),
            pltpu.VMEM((NRING, CH), jnp.int32),
            pltpu.VMEM_SHARED((ACC_ROWS, HD), jnp.float32),
            pltpu.SemaphoreType.DMA,
            pltpu.SemaphoreType.DMA,
            [pltpu.SemaphoreType.DMA] * NRING,
            [pltpu.SemaphoreType.DMA] * NRING,
        ],
    )(labels, hv, classify_weights)


def kernel(labels, hv, classify_weights):
    return _scatter_add(labels, hv, classify_weights)


# speedup vs baseline: 1.2940x; 1.0060x over previous
import jax
import jax.numpy as jnp
from jax import lax
from jax.experimental import pallas as pl
from jax.experimental.pallas import tpu as pltpu
from jax.experimental.pallas import tpu_sc as plsc

NUM_CLASSES = 100000
HD = 128
N = 16384

NC = 2
NS = 16

BLOCK = 12800
NB = 4
DUMMY = BLOCK
ACC_ROWS = BLOCK + 256
LPT = N // NS
RPT = BLOCK // NS
CH = 64
NCH = LPT // CH
NRING = 3
QS = 4
CQ = RPT // QS


def _body(labels_hbm, hv_hbm, w_hbm, out_hbm,
          labels_v, stage, dstidx, acc, sem_i, sem_o, sem_l, sem_a):
    c = lax.axis_index("c")
    s = lax.axis_index("s")
    lab_base = s * LPT

    pltpu.sync_copy(labels_hbm.at[pl.ds(lab_base, LPT)], labels_v)

    def blo_of(b):
        return (c * NB + b) * BLOCK

    def fire_init(b, q):
        row0 = blo_of(b) + s * RPT + q * CQ
        loc = s * RPT + q * CQ

        @pl.when(row0 < NUM_CLASSES)
        def _():
            pltpu.async_copy(
                w_hbm.at[pl.ds(row0, CQ)], acc.at[pl.ds(loc, CQ)], sem_i[q]
            )

    def wait_init(b, q):
        row0 = blo_of(b) + s * RPT + q * CQ
        loc = s * RPT + q * CQ

        @pl.when(row0 < NUM_CLASSES)
        def _():
            pltpu.make_async_copy(
                w_hbm.at[pl.ds(row0, CQ)], acc.at[pl.ds(loc, CQ)], sem_i[q]
            ).wait()

    def fire_copyout(b, q):
        row0 = blo_of(b) + s * RPT + q * CQ
        loc = s * RPT + q * CQ

        @pl.when(row0 < NUM_CLASSES)
        def _():
            pltpu.async_copy(
                acc.at[pl.ds(loc, CQ)], out_hbm.at[pl.ds(row0, CQ)], sem_o[q]
            )

    def wait_copyout(b, q):
        row0 = blo_of(b) + s * RPT + q * CQ
        loc = s * RPT + q * CQ

        @pl.when(row0 < NUM_CLASSES)
        def _():
            pltpu.make_async_copy(
                acc.at[pl.ds(loc, CQ)], out_hbm.at[pl.ds(row0, CQ)], sem_o[q]
            ).wait()

    def fire_load(k):
        r = k % NRING
        pltpu.async_copy(
            hv_hbm.at[pl.ds(lab_base + k * CH, CH)], stage.at[r], sem_l[r]
        )

    def wait_load(k):
        r = k % NRING
        pltpu.make_async_copy(
            hv_hbm.at[pl.ds(lab_base + k * CH, CH)], stage.at[r], sem_l[r]
        ).wait()

    def fire_add(k):
        r = k % NRING
        pltpu.async_copy(stage.at[r], acc.at[dstidx.at[r]], sem_a[r], add=True)

    def wait_add(k):
        r = k % NRING
        pltpu.make_async_copy(
            stage.at[r], acc.at[dstidx.at[r]], sem_a[r]
        ).wait()

    for q in range(QS):
        fire_init(0, q)

    for b in range(NB):
        blo = blo_of(b)
        bhi = blo + BLOCK

        for q in range(QS):
            wait_init(b, q)
        plsc.subcore_barrier()

        fire_load(0)
        fire_load(1)

        for k in range(NCH):
            r = k % NRING
            for g in range(CH // 16):
                lab = labels_v[pl.ds(k * CH + g * 16, 16)]
                in_blk = (lab >= blo) & (lab < bhi)
                dstidx[r, pl.ds(g * 16, 16)] = jnp.where(
                    in_blk, lab - blo,
                    DUMMY + ((k % 16) * 16 + lax.iota(jnp.int32, 16))
                )
            wait_load(k)
            fire_add(k)
            if k >= 1:
                wait_add(k - 1)
            if k + 2 < NCH:
                fire_load(k + 2)
        wait_add(NCH - 1)

        plsc.subcore_barrier()

        for q in range(QS):
            fire_copyout(b, q)
        for q in range(QS):
            wait_copyout(b, q)
            if b + 1 < NB:
                fire_init(b + 1, q)


@jax.jit
def _scatter_add(labels, hv, classify_weights):
    mesh = plsc.VectorSubcoreMesh(
        core_axis_name="c", subcore_axis_name="s", num_cores=NC, num_subcores=NS
    )
    return pl.kernel(
        _body,
        out_type=jax.ShapeDtypeStruct((NUM_CLASSES, HD), jnp.float32),
        mesh=mesh,
        compiler_params=pltpu.CompilerParams(needs_layout_passes=False),
        scratch_types=[
            pltpu.VMEM((LPT,), jnp.int32),
            pltpu.VMEM((NRING, CH, HD), jnp.float32),
            pltpu.VMEM((NRING, CH), jnp.int32),
            pltpu.VMEM_SHARED((ACC_ROWS, HD), jnp.float32),
            [pltpu.SemaphoreType.DMA] * QS,
            [pltpu.SemaphoreType.DMA] * QS,
            [pltpu.SemaphoreType.DMA] * NRING,
            [pltpu.SemaphoreType.DMA] * NRING,
        ],
    )(labels, hv, classify_weights)


def kernel(labels, hv, classify_weights):
    return _scatter_add(labels, hv, classify_weights)
